# baseline (speedup 1.0000x reference)
import functools

import jax
import jax.numpy as jnp
import numpy as np
from jax import lax
from jax.experimental import pallas as pl
from jax.experimental.pallas import tpu as pltpu
from jax.experimental.pallas import tpu_sc as plsc

N = 10000
E = 320000
D = 128
NG = 8
HID = 32
CUTOFF = 5.0

NC = 2
NS = 16
NW = NC * NS
LANES = 16

CHUNK = 64
DEPTH = 3
NHW = 16
BWIN = NHW * CHUNK
BWINDOWS = 10
EPT = BWIN * BWINDOWS
E_PAD = EPT * NW
IDX_ROWS = E_PAD // CHUNK

N_T = 10240
NPT = N_T // NS

MUL_UNROLL = 2

BE = 1024
FROWS = 8
NB = 2048

def _filter_body(d_ref, w1t_ref, b1_ref, w2_ref, b2_ref, f_ref):
    width = CUTOFF / (NG - 1)
    centers = (lax.broadcasted_iota(jnp.int32, (NG, 1), 0).astype(jnp.float32)
               * width)
    w1t = w1t_ref[:, :]
    w2 = w2_ref[:, :]
    b1 = b1_ref[:, :]
    b2 = b2_ref[0, :]
    for r in range(FROWS):
        d = d_ref[r, :][None, :]
        smt = jnp.exp(-0.5 * ((d - centers) / width) ** 2)
        ht = jnp.dot(w1t, smt, preferred_element_type=jnp.float32) + b1
        ht = jax.nn.softplus(ht) - jnp.log(2.0)
        f = lax.dot_general(ht, w2, (((0,), (0,)), ((), ())),
                            preferred_element_type=jnp.float32) + b2

        def rne_bf16(x):
            u = lax.bitcast_convert_type(x, jnp.uint32)
            return (u + jnp.uint32(0x7FFF)
                    + ((u >> jnp.uint32(16)) & jnp.uint32(1))) >> jnp.uint32(16)

        lo = rne_bf16(f[: BE // 2, :])
        hi = rne_bf16(f[BE // 2:, :])
        f_ref[pl.ds(r * (BE // 2), BE // 2), :] = lo | (hi << jnp.uint32(16))


_filter_call = pl.pallas_call(
    _filter_body,
    grid=(E_PAD // (FROWS * BE),),
    in_specs=[
        pl.BlockSpec((FROWS, BE), lambda i: (i, 0)),
        pl.BlockSpec((HID, NG), lambda i: (0, 0)),
        pl.BlockSpec((HID, 1), lambda i: (0, 0)),
        pl.BlockSpec((HID, D), lambda i: (0, 0)),
        pl.BlockSpec((1, D), lambda i: (0, 0)),
    ],
    out_specs=pl.BlockSpec((FROWS * BE // 2, D), lambda i: (i, 0)),
    out_shape=jax.ShapeDtypeStruct((E_PAD // 2, D), jnp.uint32),
)


def _add_body(p_ref, o_ref):
    o_ref[:, :] = p_ref[0] + p_ref[1]


_add_call = pl.pallas_call(
    _add_body,
    grid=(N_T // NB,),
    in_specs=[pl.BlockSpec((2, NB, D), lambda i: (0, i, 0))],
    out_specs=pl.BlockSpec((NB, D), lambda i: (i, 0)),
    out_shape=jax.ShapeDtypeStruct((N_T, D), jnp.float32),
)

_sc_mesh = plsc.VectorSubcoreMesh(core_axis_name="c", subcore_axis_name="s")


@functools.partial(
    pl.kernel,
    out_type=jax.ShapeDtypeStruct((NC, N_T, D), jnp.float32),
    mesh=_sc_mesh,
    scratch_types=[
        pltpu.VMEM_SHARED((N_T, D), jnp.float32),
        pltpu.VMEM((NHW, CHUNK), jnp.int32),
        pltpu.VMEM((NHW, CHUNK), jnp.int32),
        pltpu.VMEM((CHUNK, D), jnp.float32),
        pltpu.VMEM((CHUNK, D), jnp.float32),
        pltpu.VMEM((CHUNK, D), jnp.float32),
        pltpu.VMEM((CHUNK // 2, D), jnp.uint32),
        pltpu.VMEM((CHUNK // 2, D), jnp.uint32),
        pltpu.VMEM((CHUNK // 2, D), jnp.uint32),
        pltpu.SemaphoreType.DMA,
        pltpu.SemaphoreType.DMA,
        pltpu.SemaphoreType.DMA,
        pltpu.SemaphoreType.DMA,
        pltpu.SemaphoreType.DMA,
        pltpu.SemaphoreType.DMA,
        pltpu.SemaphoreType.DMA,
        pltpu.SemaphoreType.DMA,
        pltpu.SemaphoreType.DMA,
    ],
)
def _sc_conv(x_hbm, ctr_hbm, nbr_hbm, f_hbm, out_hbm,
             acc, ctr_v, nbr_v, rows0, rows1, rows2, filt0, filt1, filt2,
             gsem0, gsem1, gsem2, fsem0, fsem1, fsem2, ssem0, ssem1, ssem2):
    c = lax.axis_index("c")
    s = lax.axis_index("s")
    wid = c * NS + s

    rows = (rows0, rows1, rows2)
    filt = (filt0, filt1, filt2)
    gsem = (gsem0, gsem1, gsem2)
    fsem = (fsem0, fsem1, fsem2)
    ssem = (ssem0, ssem1, ssem2)

    zero = jnp.zeros((LANES,), jnp.float32)

    def zbody(r, carry):
        for j in range(D // LANES):
            rows0[r, pl.ds(j * LANES, LANES)] = zero
        return carry

    lax.fori_loop(0, CHUNK, zbody, 0)
    for t in range(NPT // CHUNK):
        pltpu.sync_copy(rows0.at[:],
                        acc.at[pl.ds(s * NPT + t * CHUNK, CHUNK)])

    plsc.subcore_barrier()

    tile_row0 = wid * (EPT // CHUNK)
    tile_e0 = wid * EPT

    def window(w, carry):
        r0 = tile_row0 + w * NHW
        e0 = tile_e0 + w * BWIN
        pltpu.sync_copy(ctr_hbm.at[pl.ds(r0, NHW)], ctr_v)
        pltpu.sync_copy(nbr_hbm.at[pl.ds(r0, NHW)], nbr_v)

        def issue(ch, b):
            g = pltpu.async_copy(x_hbm.at[nbr_v.at[ch]], rows[b], gsem[b])
            fo = pl.multiple_of((e0 + ch * CHUNK) // 2, CHUNK // 2)
            f = pltpu.async_copy(
                f_hbm.at[pl.ds(fo, CHUNK // 2)], filt[b], fsem[b])
            return g, f

        gd = [None] * DEPTH
        fd = [None] * DEPTH
        sd = [None] * DEPTH
        for p in range(DEPTH - 1):
            gd[p], fd[p] = issue(p, p)

        for h in range(NHW):
            b = h % DEPTH
            pf = h + DEPTH - 1
            if pf < NHW:
                pb = pf % DEPTH
                if h >= 1:
                    sd[pb].wait()
                gd[pb], fd[pb] = issue(pf, pb)
            gd[b].wait()
            fd[b].wait()

            rb = rows[b]
            fb = filt[b]

            def mbody(q, mcarry):
                r0 = 2 * q
                r1 = 2 * q + 1
                for j in range(D // LANES):
                    sl = pl.ds(j * LANES, LANES)
                    w = fb[q, sl]
                    flo = lax.bitcast_convert_type(
                        w << jnp.uint32(16), jnp.float32)
                    fhi = lax.bitcast_convert_type(
                        w & jnp.uint32(0xFFFF0000), jnp.float32)
                    rb[r0, sl] = rb[r0, sl] * flo
                    rb[r1, sl] = rb[r1, sl] * fhi
                return mcarry

            lax.fori_loop(0, CHUNK // 2, mbody, 0)

            sd[b] = pltpu.async_copy(rb, acc.at[ctr_v.at[h]], ssem[b],
                                     add=True)
        for k in range(1, DEPTH):
            sd[(NHW - k) % DEPTH].wait()
        return carry

    lax.fori_loop(0, BWINDOWS, window, 0)

    plsc.subcore_barrier()
    pltpu.sync_copy(acc.at[pl.ds(s * NPT, NPT)],
                    out_hbm.at[c, pl.ds(s * NPT, NPT)])


def kernel(channels, edge_distances, edge_index, W1, b1, W2, b2):
    npad = E_PAD - E
    d_pad = jnp.concatenate([edge_distances, jnp.zeros((npad,), jnp.float32)])
    pad_i = jnp.arange(npad, dtype=jnp.int32)
    ctr = jnp.concatenate([edge_index[0], N + (pad_i % NS)])
    nbr = jnp.concatenate([edge_index[1], pad_i % NS])
    ctr = ctr.reshape(-1, 2, BE // 2).transpose(0, 2, 1).reshape(-1)
    nbr = nbr.reshape(-1, 2, BE // 2).transpose(0, 2, 1).reshape(-1)
    ctr2 = ctr.reshape(IDX_ROWS, CHUNK)
    nbr2 = nbr.reshape(IDX_ROWS, CHUNK)

    f_edge = _filter_call(d_pad.reshape(E_PAD // BE, BE), W1.T,
                          b1.reshape(HID, 1), W2, b2.reshape(1, D))
    partial = _sc_conv(channels, ctr2, nbr2, f_edge)
    return _add_call(partial)[:N]


# speedup vs baseline: 1.0181x; 1.0181x over previous
import functools

import jax
import jax.numpy as jnp
import numpy as np
from jax import lax
from jax.experimental import pallas as pl
from jax.experimental.pallas import tpu as pltpu
from jax.experimental.pallas import tpu_sc as plsc

N = 10000
E = 320000
D = 128
NG = 8
HID = 32
CUTOFF = 5.0

NC = 2
NS = 16
NW = NC * NS
LANES = 16

CHUNK = 64
DEPTH = 2
NHW = 16
BWIN = NHW * CHUNK
BWINDOWS = 10
EPT = BWIN * BWINDOWS
E_PAD = EPT * NW
IDX_ROWS = E_PAD // CHUNK

N_T = 10240
NPT = N_T // NS

MUL_UNROLL = 2

BE = 1024
FROWS = 8
NB = 2048

def _filter_body(d_ref, w1t_ref, b1_ref, w2_ref, b2_ref, f_ref):
    width = CUTOFF / (NG - 1)
    centers = (lax.broadcasted_iota(jnp.int32, (NG, 1), 0).astype(jnp.float32)
               * width)
    w1t = w1t_ref[:, :]
    w2 = w2_ref[:, :]
    b1 = b1_ref[:, :]
    b2 = b2_ref[0, :]
    for r in range(FROWS):
        d = d_ref[r, :][None, :]
        smt = jnp.exp(-0.5 * ((d - centers) / width) ** 2)
        ht = jnp.dot(w1t, smt, preferred_element_type=jnp.float32) + b1
        ht = jax.nn.softplus(ht) - jnp.log(2.0)
        f = lax.dot_general(ht, w2, (((0,), (0,)), ((), ())),
                            preferred_element_type=jnp.float32) + b2

        def rne_bf16(x):
            u = lax.bitcast_convert_type(x, jnp.uint32)
            return (u + jnp.uint32(0x7FFF)
                    + ((u >> jnp.uint32(16)) & jnp.uint32(1))) >> jnp.uint32(16)

        lo = rne_bf16(f[: BE // 2, :])
        hi = rne_bf16(f[BE // 2:, :])
        f_ref[pl.ds(r * (BE // 2), BE // 2), :] = lo | (hi << jnp.uint32(16))


_filter_call = pl.pallas_call(
    _filter_body,
    grid=(E_PAD // (FROWS * BE),),
    in_specs=[
        pl.BlockSpec((FROWS, BE), lambda i: (i, 0)),
        pl.BlockSpec((HID, NG), lambda i: (0, 0)),
        pl.BlockSpec((HID, 1), lambda i: (0, 0)),
        pl.BlockSpec((HID, D), lambda i: (0, 0)),
        pl.BlockSpec((1, D), lambda i: (0, 0)),
    ],
    out_specs=pl.BlockSpec((FROWS * BE // 2, D), lambda i: (i, 0)),
    out_shape=jax.ShapeDtypeStruct((E_PAD // 2, D), jnp.uint32),
)


def _add_body(p_ref, o_ref):
    o_ref[:, :] = p_ref[0] + p_ref[1]


_add_call = pl.pallas_call(
    _add_body,
    grid=(N_T // NB,),
    in_specs=[pl.BlockSpec((2, NB, D), lambda i: (0, i, 0))],
    out_specs=pl.BlockSpec((NB, D), lambda i: (i, 0)),
    out_shape=jax.ShapeDtypeStruct((N_T, D), jnp.float32),
)

_sc_mesh = plsc.VectorSubcoreMesh(core_axis_name="c", subcore_axis_name="s")


@functools.partial(
    pl.kernel,
    out_type=jax.ShapeDtypeStruct((NC, N_T, D), jnp.float32),
    mesh=_sc_mesh,
    scratch_types=[
        pltpu.VMEM_SHARED((N_T, D), jnp.float32),
        pltpu.VMEM((NHW, CHUNK), jnp.int32),
        pltpu.VMEM((NHW, CHUNK), jnp.int32),
        pltpu.VMEM((CHUNK, D), jnp.float32),
        pltpu.VMEM((CHUNK, D), jnp.float32),
        pltpu.VMEM((CHUNK // 2, D), jnp.uint32),
        pltpu.VMEM((CHUNK // 2, D), jnp.uint32),
        pltpu.SemaphoreType.DMA,
        pltpu.SemaphoreType.DMA,
        pltpu.SemaphoreType.DMA,
        pltpu.SemaphoreType.DMA,
        pltpu.SemaphoreType.DMA,
        pltpu.SemaphoreType.DMA,
    ],
)
def _sc_conv(x_hbm, ctr_hbm, nbr_hbm, f_hbm, out_hbm,
             acc, ctr_v, nbr_v, rows0, rows1, filt0, filt1,
             gsem0, gsem1, fsem0, fsem1, ssem0, ssem1):
    c = lax.axis_index("c")
    s = lax.axis_index("s")
    wid = c * NS + s

    rows = (rows0, rows1)
    filt = (filt0, filt1)
    gsem = (gsem0, gsem1)
    fsem = (fsem0, fsem1)
    ssem = (ssem0, ssem1)

    zero = jnp.zeros((LANES,), jnp.float32)

    def zbody(r, carry):
        for j in range(D // LANES):
            rows0[r, pl.ds(j * LANES, LANES)] = zero
        return carry

    lax.fori_loop(0, CHUNK, zbody, 0)
    for t in range(NPT // CHUNK):
        pltpu.sync_copy(rows0.at[:],
                        acc.at[pl.ds(s * NPT + t * CHUNK, CHUNK)])

    plsc.subcore_barrier()

    tile_row0 = wid * (EPT // CHUNK)
    tile_e0 = wid * EPT

    def window(w, carry):
        r0 = tile_row0 + w * NHW
        e0 = tile_e0 + w * BWIN
        pltpu.sync_copy(ctr_hbm.at[pl.ds(r0, NHW)], ctr_v)
        pltpu.sync_copy(nbr_hbm.at[pl.ds(r0, NHW)], nbr_v)

        def issue(ch, b):
            g = pltpu.async_copy(x_hbm.at[nbr_v.at[ch]], rows[b], gsem[b])
            fo = pl.multiple_of((e0 + ch * CHUNK) // 2, CHUNK // 2)
            f = pltpu.async_copy(
                f_hbm.at[pl.ds(fo, CHUNK // 2)], filt[b], fsem[b])
            return g, f

        gd = [None] * DEPTH
        fd = [None] * DEPTH
        sd = [None] * DEPTH
        for p in range(DEPTH - 1):
            gd[p], fd[p] = issue(p, p)

        for h in range(NHW):
            b = h % DEPTH
            pf = h + DEPTH - 1
            if pf < NHW:
                pb = pf % DEPTH
                if h >= 1:
                    sd[pb].wait()
                gd[pb], fd[pb] = issue(pf, pb)
            gd[b].wait()
            fd[b].wait()

            rb = rows[b]
            fb = filt[b]

            def mbody(q, mcarry):
                r0 = 2 * q
                r1 = 2 * q + 1
                for j in range(D // LANES):
                    sl = pl.ds(j * LANES, LANES)
                    w = fb[q, sl]
                    flo = lax.bitcast_convert_type(
                        w << jnp.uint32(16), jnp.float32)
                    fhi = lax.bitcast_convert_type(
                        w & jnp.uint32(0xFFFF0000), jnp.float32)
                    rb[r0, sl] = rb[r0, sl] * flo
                    rb[r1, sl] = rb[r1, sl] * fhi
                return mcarry

            lax.fori_loop(0, CHUNK // 2, mbody, 0)

            sd[b] = pltpu.async_copy(rb, acc.at[ctr_v.at[h]], ssem[b],
                                     add=True)
        for k in range(1, DEPTH):
            sd[(NHW - k) % DEPTH].wait()
        return carry

    lax.fori_loop(0, BWINDOWS, window, 0)

    plsc.subcore_barrier()
    pltpu.sync_copy(acc.at[pl.ds(s * NPT, NPT)],
                    out_hbm.at[c, pl.ds(s * NPT, NPT)])


def kernel(channels, edge_distances, edge_index, W1, b1, W2, b2):
    npad = E_PAD - E
    d_pad = jnp.concatenate([edge_distances, jnp.zeros((npad,), jnp.float32)])
    pad_i = jnp.arange(npad, dtype=jnp.int32)
    ctr = jnp.concatenate([edge_index[0], N + (pad_i % NS)])
    nbr = jnp.concatenate([edge_index[1], pad_i % NS])
    ctr = ctr.reshape(-1, 2, BE // 2).transpose(0, 2, 1).reshape(-1)
    nbr = nbr.reshape(-1, 2, BE // 2).transpose(0, 2, 1).reshape(-1)
    ctr2 = ctr.reshape(IDX_ROWS, CHUNK)
    nbr2 = nbr.reshape(IDX_ROWS, CHUNK)

    f_edge = _filter_call(d_pad.reshape(E_PAD // BE, BE), W1.T,
                          b1.reshape(HID, 1), W2, b2.reshape(1, D))
    partial = _sc_conv(channels, ctr2, nbr2, f_edge)
    return _add_call(partial)[:N]


# speedup vs baseline: 1.2335x; 1.2116x over previous
import functools

import jax
import jax.numpy as jnp
import numpy as np
from jax import lax
from jax.experimental import pallas as pl
from jax.experimental.pallas import tpu as pltpu
from jax.experimental.pallas import tpu_sc as plsc

N = 10000
E = 320000
D = 128
NG = 8
HID = 32
CUTOFF = 5.0

NC = 2
NS = 16
NW = NC * NS
LANES = 16

CHUNK = 64
DEPTH = 2
NHW = 16
BWIN = NHW * CHUNK
BWINDOWS = 10
EPT = BWIN * BWINDOWS
E_PAD = EPT * NW
IDX_ROWS = E_PAD // CHUNK

N_T = 10240
NPT = N_T // NS

MUL_UNROLL = 2

BE = 1024
FROWS = 8
NB = 2048

def _filter_body(d_ref, w1t_ref, b1_ref, w2_ref, b2_ref, f_ref):
    width = CUTOFF / (NG - 1)
    centers = (lax.broadcasted_iota(jnp.int32, (NG, 1), 0).astype(jnp.float32)
               * width)
    w1t = w1t_ref[:, :]
    w2 = w2_ref[:, :]
    b1 = b1_ref[:, :]
    b2 = b2_ref[0, :]
    for r in range(FROWS):
        d = d_ref[r, :][None, :]
        smt = jnp.exp(-0.5 * ((d - centers) / width) ** 2)
        ht = jnp.dot(w1t, smt, preferred_element_type=jnp.float32) + b1
        ht = jax.nn.softplus(ht) - jnp.log(2.0)
        f = lax.dot_general(ht, w2, (((0,), (0,)), ((), ())),
                            preferred_element_type=jnp.float32) + b2

        def rne_bf16(x):
            u = lax.bitcast_convert_type(x, jnp.uint32)
            return (u + jnp.uint32(0x7FFF)
                    + ((u >> jnp.uint32(16)) & jnp.uint32(1))) >> jnp.uint32(16)

        lo = rne_bf16(f[: BE // 2, :])
        hi = rne_bf16(f[BE // 2:, :])
        f_ref[pl.ds(r * (BE // 2), BE // 2), :] = lo | (hi << jnp.uint32(16))


_filter_call = pl.pallas_call(
    _filter_body,
    grid=(E_PAD // (FROWS * BE),),
    in_specs=[
        pl.BlockSpec((FROWS, BE), lambda i: (i, 0)),
        pl.BlockSpec((HID, NG), lambda i: (0, 0)),
        pl.BlockSpec((HID, 1), lambda i: (0, 0)),
        pl.BlockSpec((HID, D), lambda i: (0, 0)),
        pl.BlockSpec((1, D), lambda i: (0, 0)),
    ],
    out_specs=pl.BlockSpec((FROWS * BE // 2, D), lambda i: (i, 0)),
    out_shape=jax.ShapeDtypeStruct((E_PAD // 2, D), jnp.uint32),
)


def _add_body(p_ref, o_ref):
    o_ref[:, :] = p_ref[0] + p_ref[1]


_add_call = pl.pallas_call(
    _add_body,
    grid=(N_T // NB,),
    in_specs=[pl.BlockSpec((2, NB, D), lambda i: (0, i, 0))],
    out_specs=pl.BlockSpec((NB, D), lambda i: (i, 0)),
    out_shape=jax.ShapeDtypeStruct((N_T, D), jnp.float32),
)

_sc_mesh = plsc.VectorSubcoreMesh(core_axis_name="c", subcore_axis_name="s")


@functools.partial(
    pl.kernel,
    out_type=jax.ShapeDtypeStruct((NC, N_T, D), jnp.float32),
    mesh=_sc_mesh,
    scratch_types=[
        pltpu.VMEM_SHARED((N_T, D), jnp.float32),
        pltpu.VMEM((NHW, CHUNK), jnp.int32),
        pltpu.VMEM((NHW, CHUNK), jnp.int32),
        pltpu.VMEM((CHUNK, D), jnp.float32),
        pltpu.VMEM((CHUNK, D), jnp.float32),
        pltpu.VMEM((CHUNK // 2, D), jnp.uint32),
        pltpu.VMEM((CHUNK // 2, D), jnp.uint32),
        pltpu.SemaphoreType.DMA,
        pltpu.SemaphoreType.DMA,
        pltpu.SemaphoreType.DMA,
        pltpu.SemaphoreType.DMA,
        pltpu.SemaphoreType.DMA,
        pltpu.SemaphoreType.DMA,
    ],
)
def _sc_conv(x_hbm, ctr_hbm, nbr_hbm, f_hbm, out_hbm,
             acc, ctr_v, nbr_v, rows0, rows1, filt0, filt1,
             gsem0, gsem1, fsem0, fsem1, ssem0, ssem1):
    c = lax.axis_index("c")
    s = lax.axis_index("s")
    wid = c * NS + s

    rows = (rows0, rows1)
    filt = (filt0, filt1)
    gsem = (gsem0, gsem1)
    fsem = (fsem0, fsem1)
    ssem = (ssem0, ssem1)

    zero = jnp.zeros((LANES,), jnp.float32)

    def zbody(r, carry):
        for j in range(D // LANES):
            rows0[r, pl.ds(j * LANES, LANES)] = zero
        return carry

    lax.fori_loop(0, CHUNK, zbody, 0)
    for t in range(NPT // CHUNK):
        pltpu.sync_copy(rows0.at[:],
                        acc.at[pl.ds(s * NPT + t * CHUNK, CHUNK)])

    plsc.subcore_barrier()

    tile_row0 = wid * (EPT // CHUNK)
    tile_e0 = wid * EPT

    def window(w, carry):
        r0 = tile_row0 + w * NHW
        e0 = tile_e0 + w * BWIN
        pltpu.sync_copy(ctr_hbm.at[pl.ds(r0, NHW)], ctr_v)
        pltpu.sync_copy(nbr_hbm.at[pl.ds(r0, NHW)], nbr_v)

        def issue(ch, b):
            g = pltpu.async_copy(x_hbm.at[nbr_v.at[ch]], rows[b], gsem[b])
            fo = pl.multiple_of((e0 + ch * CHUNK) // 2, CHUNK // 2)
            f = pltpu.async_copy(
                f_hbm.at[pl.ds(fo, CHUNK // 2)], filt[b], fsem[b])
            return g, f

        gd = [None] * DEPTH
        fd = [None] * DEPTH
        sd = [None] * DEPTH
        for p in range(DEPTH - 1):
            gd[p], fd[p] = issue(p, p)

        for h in range(NHW):
            b = h % DEPTH
            pf = h + DEPTH - 1
            if pf < NHW:
                pb = pf % DEPTH
                if h >= 1:
                    sd[pb].wait()
                gd[pb], fd[pb] = issue(pf, pb)
            gd[b].wait()
            fd[b].wait()

            rb = rows[b]
            fb = filt[b]

            def mbody(q, mcarry):
                r0 = 2 * q
                r1 = 2 * q + 1
                ws = [fb[q, pl.ds(j * LANES, LANES)]
                      for j in range(D // LANES)]
                flo = [lax.bitcast_convert_type(w << jnp.uint32(16),
                                                jnp.float32) for w in ws]
                fhi = [lax.bitcast_convert_type(w & jnp.uint32(0xFFFF0000),
                                                jnp.float32) for w in ws]
                for j in range(D // LANES):
                    sl = pl.ds(j * LANES, LANES)
                    rb[r0, sl] = rb[r0, sl] * flo[j]
                for j in range(D // LANES):
                    sl = pl.ds(j * LANES, LANES)
                    rb[r1, sl] = rb[r1, sl] * fhi[j]
                return mcarry

            lax.fori_loop(0, CHUNK // 2, mbody, 0)

            sd[b] = pltpu.async_copy(rb, acc.at[ctr_v.at[h]], ssem[b],
                                     add=True)
        for k in range(1, DEPTH):
            sd[(NHW - k) % DEPTH].wait()
        return carry

    lax.fori_loop(0, BWINDOWS, window, 0)

    plsc.subcore_barrier()
    pltpu.sync_copy(acc.at[pl.ds(s * NPT, NPT)],
                    out_hbm.at[c, pl.ds(s * NPT, NPT)])


def kernel(channels, edge_distances, edge_index, W1, b1, W2, b2):
    npad = E_PAD - E
    d_pad = jnp.concatenate([edge_distances, jnp.zeros((npad,), jnp.float32)])
    pad_i = jnp.arange(npad, dtype=jnp.int32)
    ctr = jnp.concatenate([edge_index[0], N + (pad_i % NS)])
    nbr = jnp.concatenate([edge_index[1], pad_i % NS])
    ctr = ctr.reshape(-1, 2, BE // 2).transpose(0, 2, 1).reshape(-1)
    nbr = nbr.reshape(-1, 2, BE // 2).transpose(0, 2, 1).reshape(-1)
    ctr2 = ctr.reshape(IDX_ROWS, CHUNK)
    nbr2 = nbr.reshape(IDX_ROWS, CHUNK)

    f_edge = _filter_call(d_pad.reshape(E_PAD // BE, BE), W1.T,
                          b1.reshape(HID, 1), W2, b2.reshape(1, D))
    partial = _sc_conv(channels, ctr2, nbr2, f_edge)
    return _add_call(partial)[:N]


# speedup vs baseline: 1.9540x; 1.5841x over previous
import functools

import jax
import jax.numpy as jnp
import numpy as np
from jax import lax
from jax.experimental import pallas as pl
from jax.experimental.pallas import tpu as pltpu
from jax.experimental.pallas import tpu_sc as plsc

N = 10000
E = 320000
D = 128
NG = 8
HID = 32
CUTOFF = 5.0

NC = 2
NS = 16
NW = NC * NS
LANES = 16

CHUNK = 64
DEPTH = 2
NHW = 16
BWIN = NHW * CHUNK
BWINDOWS = 10
EPT = BWIN * BWINDOWS
E_PAD = EPT * NW
IDX_ROWS = E_PAD // CHUNK

N_T = 10240
NPT = N_T // NS

MUL_UNROLL = 2

BE = 1024
FROWS = 8
NB = 2048

def _filter_body(d_ref, w1t_ref, b1_ref, w2_ref, b2_ref, f_ref):
    width = CUTOFF / (NG - 1)
    centers = (lax.broadcasted_iota(jnp.int32, (NG, 1), 0).astype(jnp.float32)
               * width)
    w1t = w1t_ref[:, :]
    w2 = w2_ref[:, :]
    b1 = b1_ref[:, :]
    b2 = b2_ref[0, :]
    for r in range(FROWS):
        d = d_ref[r, :][None, :]
        smt = jnp.exp(-0.5 * ((d - centers) / width) ** 2)
        ht = jnp.dot(w1t, smt, preferred_element_type=jnp.float32) + b1
        ht = jax.nn.softplus(ht) - jnp.log(2.0)
        f = lax.dot_general(ht, w2, (((0,), (0,)), ((), ())),
                            preferred_element_type=jnp.float32) + b2

        def rne_bf16(x):
            u = lax.bitcast_convert_type(x, jnp.uint32)
            return (u + jnp.uint32(0x7FFF)
                    + ((u >> jnp.uint32(16)) & jnp.uint32(1))) >> jnp.uint32(16)

        f2 = f.reshape(BE // 128, 2, 64, D)
        lo = rne_bf16(f2[:, 0].reshape(BE // 2, D))
        hi = rne_bf16(f2[:, 1].reshape(BE // 2, D))
        f_ref[pl.ds(r * (BE // 2), BE // 2), :] = lo | (hi << jnp.uint32(16))


_filter_call = pl.pallas_call(
    _filter_body,
    grid=(E_PAD // (FROWS * BE),),
    in_specs=[
        pl.BlockSpec((FROWS, BE), lambda i: (i, 0)),
        pl.BlockSpec((HID, NG), lambda i: (0, 0)),
        pl.BlockSpec((HID, 1), lambda i: (0, 0)),
        pl.BlockSpec((HID, D), lambda i: (0, 0)),
        pl.BlockSpec((1, D), lambda i: (0, 0)),
    ],
    out_specs=pl.BlockSpec((FROWS * BE // 2, D), lambda i: (i, 0)),
    out_shape=jax.ShapeDtypeStruct((E_PAD // 2, D), jnp.uint32),
)


def _add_body(p_ref, o_ref):
    o_ref[:, :] = p_ref[0] + p_ref[1]


_add_call = pl.pallas_call(
    _add_body,
    grid=(N_T // NB,),
    in_specs=[pl.BlockSpec((2, NB, D), lambda i: (0, i, 0))],
    out_specs=pl.BlockSpec((NB, D), lambda i: (i, 0)),
    out_shape=jax.ShapeDtypeStruct((N_T, D), jnp.float32),
)

_sc_mesh = plsc.VectorSubcoreMesh(core_axis_name="c", subcore_axis_name="s")


@functools.partial(
    pl.kernel,
    out_type=jax.ShapeDtypeStruct((NC, N_T, D), jnp.float32),
    mesh=_sc_mesh,
    scratch_types=[
        pltpu.VMEM_SHARED((N_T, D), jnp.float32),
        pltpu.VMEM((NHW, CHUNK), jnp.int32),
        pltpu.VMEM((NHW, CHUNK), jnp.int32),
        pltpu.VMEM((CHUNK, D), jnp.float32),
        pltpu.VMEM((CHUNK, D), jnp.float32),
        pltpu.VMEM((CHUNK, D), jnp.uint32),
        pltpu.VMEM((CHUNK, D), jnp.uint32),
        pltpu.SemaphoreType.DMA,
        pltpu.SemaphoreType.DMA,
        pltpu.SemaphoreType.DMA,
        pltpu.SemaphoreType.DMA,
        pltpu.SemaphoreType.DMA,
        pltpu.SemaphoreType.DMA,
    ],
)
def _sc_conv(x_hbm, ctr_hbm, nbr_hbm, f_hbm, out_hbm,
             acc, ctr_v, nbr_v, rows0, rows1, filt0, filt1,
             gsem0, gsem1, fsem0, fsem1, ssem0, ssem1):
    c = lax.axis_index("c")
    s = lax.axis_index("s")
    wid = c * NS + s

    rows = (rows0, rows1)
    filt = (filt0, filt1)
    gsem = (gsem0, gsem1)
    fsem = (fsem0, fsem1)
    ssem = (ssem0, ssem1)

    zero = jnp.zeros((LANES,), jnp.float32)

    def zbody(r, carry):
        for j in range(D // LANES):
            rows0[r, pl.ds(j * LANES, LANES)] = zero
        return carry

    lax.fori_loop(0, CHUNK, zbody, 0)
    for t in range(NPT // CHUNK):
        pltpu.sync_copy(rows0.at[:],
                        acc.at[pl.ds(s * NPT + t * CHUNK, CHUNK)])

    plsc.subcore_barrier()

    tile_row0 = wid * (EPT // CHUNK)
    tile_e0 = wid * EPT

    def window(w, carry):
        r0 = tile_row0 + w * NHW
        e0 = tile_e0 + w * BWIN
        pltpu.sync_copy(ctr_hbm.at[pl.ds(r0, NHW)], ctr_v)
        pltpu.sync_copy(nbr_hbm.at[pl.ds(r0, NHW)], nbr_v)

        def issue(ch, b):
            g = pltpu.async_copy(x_hbm.at[nbr_v.at[ch]], rows[b], gsem[b])
            f = None
            if ch % 2 == 0:
                fb = (ch // 2) % 2
                fo = pl.multiple_of((e0 + ch * CHUNK) // 2, CHUNK)
                f = pltpu.async_copy(
                    f_hbm.at[pl.ds(fo, CHUNK)], filt[fb], fsem[fb])
            return g, f

        gd = [None] * DEPTH
        fd = [None, None]
        sd = [None] * DEPTH
        for p in range(DEPTH - 1):
            gd[p], f0 = issue(p, p)
            if f0 is not None:
                fd[(p // 2) % 2] = f0

        for h in range(NHW):
            b = h % DEPTH
            pf = h + DEPTH - 1
            if pf < NHW:
                pb = pf % DEPTH
                if h >= 1:
                    sd[pb].wait()
                gd[pb], f0 = issue(pf, pb)
                if f0 is not None:
                    fd[(pf // 2) % 2] = f0
            gd[b].wait()
            if h % 2 == 0:
                fd[(h // 2) % 2].wait()

            rb = rows[b]
            fb = filt[(h // 2) % 2]

            use_hi = h % 2 == 1

            def mbody(q, mcarry):
                r0 = 2 * q
                r1 = 2 * q + 1
                ws = [fb[rr, pl.ds(j * LANES, LANES)]
                      for rr in (r0, r1) for j in range(D // LANES)]
                if use_hi:
                    fv = [lax.bitcast_convert_type(
                        w & jnp.uint32(0xFFFF0000), jnp.float32) for w in ws]
                else:
                    fv = [lax.bitcast_convert_type(
                        w << jnp.uint32(16), jnp.float32) for w in ws]
                for k, rr in enumerate((r0, r1)):
                    for j in range(D // LANES):
                        sl = pl.ds(j * LANES, LANES)
                        rb[rr, sl] = rb[rr, sl] * fv[k * (D // LANES) + j]
                return mcarry

            lax.fori_loop(0, CHUNK // 2, mbody, 0)

            sd[b] = pltpu.async_copy(rb, acc.at[ctr_v.at[h]], ssem[b],
                                     add=True)
        for k in range(1, DEPTH):
            sd[(NHW - k) % DEPTH].wait()
        return carry

    lax.fori_loop(0, BWINDOWS, window, 0)

    plsc.subcore_barrier()
    pltpu.sync_copy(acc.at[pl.ds(s * NPT, NPT)],
                    out_hbm.at[c, pl.ds(s * NPT, NPT)])


def kernel(channels, edge_distances, edge_index, W1, b1, W2, b2):
    npad = E_PAD - E
    d_pad = jnp.concatenate([edge_distances, jnp.zeros((npad,), jnp.float32)])
    pad_i = jnp.arange(npad, dtype=jnp.int32)
    ctr = jnp.concatenate([edge_index[0], N + (pad_i % NS)])
    nbr = jnp.concatenate([edge_index[1], pad_i % NS])
    ctr2 = ctr.reshape(IDX_ROWS, CHUNK)
    nbr2 = nbr.reshape(IDX_ROWS, CHUNK)

    f_edge = _filter_call(d_pad.reshape(E_PAD // BE, BE), W1.T,
                          b1.reshape(HID, 1), W2, b2.reshape(1, D))
    partial = _sc_conv(channels, ctr2, nbr2, f_edge)
    return _add_call(partial)[:N]


# speedup vs baseline: 2.0261x; 1.0369x over previous
import functools

import jax
import jax.numpy as jnp
import numpy as np
from jax import lax
from jax.experimental import pallas as pl
from jax.experimental.pallas import tpu as pltpu
from jax.experimental.pallas import tpu_sc as plsc

N = 10000
E = 320000
D = 128
NG = 8
HID = 32
CUTOFF = 5.0

NC = 2
NS = 16
NW = NC * NS
LANES = 16

CHUNK = 64
DEPTH = 3
NHW = 16
BWIN = NHW * CHUNK
BWINDOWS = 10
EPT = BWIN * BWINDOWS
E_PAD = EPT * NW
IDX_ROWS = E_PAD // CHUNK

N_T = 10240
NPT = N_T // NS

MUL_UNROLL = 2

BE = 1024
FROWS = 8
NB = 2048

def _filter_body(d_ref, w1t_ref, b1_ref, w2_ref, b2_ref, f_ref):
    width = CUTOFF / (NG - 1)
    centers = (lax.broadcasted_iota(jnp.int32, (NG, 1), 0).astype(jnp.float32)
               * width)
    w1t = w1t_ref[:, :]
    w2 = w2_ref[:, :]
    b1 = b1_ref[:, :]
    b2 = b2_ref[0, :]
    for r in range(FROWS):
        d = d_ref[r, :][None, :]
        smt = jnp.exp(-0.5 * ((d - centers) / width) ** 2)
        ht = jnp.dot(w1t, smt, preferred_element_type=jnp.float32) + b1
        ht = jax.nn.softplus(ht) - jnp.log(2.0)
        f = lax.dot_general(ht, w2, (((0,), (0,)), ((), ())),
                            preferred_element_type=jnp.float32) + b2

        def rne_bf16(x):
            u = lax.bitcast_convert_type(x, jnp.uint32)
            return (u + jnp.uint32(0x7FFF)
                    + ((u >> jnp.uint32(16)) & jnp.uint32(1))) >> jnp.uint32(16)

        f2 = f.reshape(BE // 128, 2, 64, D)
        lo = rne_bf16(f2[:, 0].reshape(BE // 2, D))
        hi = rne_bf16(f2[:, 1].reshape(BE // 2, D))
        f_ref[pl.ds(r * (BE // 2), BE // 2), :] = lo | (hi << jnp.uint32(16))


_filter_call = pl.pallas_call(
    _filter_body,
    grid=(E_PAD // (FROWS * BE),),
    in_specs=[
        pl.BlockSpec((FROWS, BE), lambda i: (i, 0)),
        pl.BlockSpec((HID, NG), lambda i: (0, 0)),
        pl.BlockSpec((HID, 1), lambda i: (0, 0)),
        pl.BlockSpec((HID, D), lambda i: (0, 0)),
        pl.BlockSpec((1, D), lambda i: (0, 0)),
    ],
    out_specs=pl.BlockSpec((FROWS * BE // 2, D), lambda i: (i, 0)),
    out_shape=jax.ShapeDtypeStruct((E_PAD // 2, D), jnp.uint32),
)


def _add_body(p_ref, o_ref):
    o_ref[:, :] = p_ref[0] + p_ref[1]


_add_call = pl.pallas_call(
    _add_body,
    grid=(N_T // NB,),
    in_specs=[pl.BlockSpec((2, NB, D), lambda i: (0, i, 0))],
    out_specs=pl.BlockSpec((NB, D), lambda i: (i, 0)),
    out_shape=jax.ShapeDtypeStruct((N_T, D), jnp.float32),
)

_sc_mesh = plsc.VectorSubcoreMesh(core_axis_name="c", subcore_axis_name="s")


@functools.partial(
    pl.kernel,
    out_type=jax.ShapeDtypeStruct((NC, N_T, D), jnp.float32),
    mesh=_sc_mesh,
    scratch_types=[
        pltpu.VMEM_SHARED((N_T, D), jnp.float32),
        pltpu.VMEM((NHW, CHUNK), jnp.int32),
        pltpu.VMEM((NHW, CHUNK), jnp.int32),
        pltpu.VMEM((CHUNK, D), jnp.float32),
        pltpu.VMEM((CHUNK, D), jnp.float32),
        pltpu.VMEM((CHUNK, D), jnp.float32),
        pltpu.VMEM((CHUNK, D), jnp.uint32),
        pltpu.VMEM((CHUNK, D), jnp.uint32),
        pltpu.SemaphoreType.DMA,
        pltpu.SemaphoreType.DMA,
        pltpu.SemaphoreType.DMA,
        pltpu.SemaphoreType.DMA,
        pltpu.SemaphoreType.DMA,
        pltpu.SemaphoreType.DMA,
        pltpu.SemaphoreType.DMA,
        pltpu.SemaphoreType.DMA,
    ],
)
def _sc_conv(x_hbm, ctr_hbm, nbr_hbm, f_hbm, out_hbm,
             acc, ctr_v, nbr_v, rows0, rows1, rows2, filt0, filt1,
             gsem0, gsem1, gsem2, fsem0, fsem1, ssem0, ssem1, ssem2):
    c = lax.axis_index("c")
    s = lax.axis_index("s")
    wid = c * NS + s

    rows = (rows0, rows1, rows2)
    filt = (filt0, filt1)
    gsem = (gsem0, gsem1, gsem2)
    fsem = (fsem0, fsem1)
    ssem = (ssem0, ssem1, ssem2)

    zero = jnp.zeros((LANES,), jnp.float32)

    def zbody(r, carry):
        for j in range(D // LANES):
            rows0[r, pl.ds(j * LANES, LANES)] = zero
        return carry

    lax.fori_loop(0, CHUNK, zbody, 0)
    for t in range(NPT // CHUNK):
        pltpu.sync_copy(rows0.at[:],
                        acc.at[pl.ds(s * NPT + t * CHUNK, CHUNK)])

    plsc.subcore_barrier()

    tile_row0 = wid * (EPT // CHUNK)
    tile_e0 = wid * EPT

    def window(w, carry):
        r0 = tile_row0 + w * NHW
        e0 = tile_e0 + w * BWIN
        pltpu.sync_copy(ctr_hbm.at[pl.ds(r0, NHW)], ctr_v)
        pltpu.sync_copy(nbr_hbm.at[pl.ds(r0, NHW)], nbr_v)

        def issue(ch, b):
            g = pltpu.async_copy(x_hbm.at[nbr_v.at[ch]], rows[b], gsem[b])
            f = None
            if ch % 2 == 0:
                fb = (ch // 2) % 2
                fo = pl.multiple_of((e0 + ch * CHUNK) // 2, CHUNK)
                f = pltpu.async_copy(
                    f_hbm.at[pl.ds(fo, CHUNK)], filt[fb], fsem[fb])
            return g, f

        gd = [None] * DEPTH
        fd = [None, None]
        sd = [None] * DEPTH
        for p in range(DEPTH - 1):
            gd[p], f0 = issue(p, p)
            if f0 is not None:
                fd[(p // 2) % 2] = f0

        for h in range(NHW):
            b = h % DEPTH
            pf = h + DEPTH - 1
            if pf < NHW:
                pb = pf % DEPTH
                if h >= 1:
                    sd[pb].wait()
                gd[pb], f0 = issue(pf, pb)
                if f0 is not None:
                    fd[(pf // 2) % 2] = f0
            gd[b].wait()
            if h % 2 == 0:
                fd[(h // 2) % 2].wait()

            rb = rows[b]
            fb = filt[(h // 2) % 2]

            use_hi = h % 2 == 1

            def mbody(q, mcarry):
                r0 = 2 * q
                r1 = 2 * q + 1
                ws = [fb[rr, pl.ds(j * LANES, LANES)]
                      for rr in (r0, r1) for j in range(D // LANES)]
                if use_hi:
                    fv = [lax.bitcast_convert_type(
                        w & jnp.uint32(0xFFFF0000), jnp.float32) for w in ws]
                else:
                    fv = [lax.bitcast_convert_type(
                        w << jnp.uint32(16), jnp.float32) for w in ws]
                for k, rr in enumerate((r0, r1)):
                    for j in range(D // LANES):
                        sl = pl.ds(j * LANES, LANES)
                        rb[rr, sl] = rb[rr, sl] * fv[k * (D // LANES) + j]
                return mcarry

            lax.fori_loop(0, CHUNK // 2, mbody, 0)

            sd[b] = pltpu.async_copy(rb, acc.at[ctr_v.at[h]], ssem[b],
                                     add=True)
        for k in range(1, DEPTH):
            sd[(NHW - k) % DEPTH].wait()
        return carry

    lax.fori_loop(0, BWINDOWS, window, 0)

    plsc.subcore_barrier()
    pltpu.sync_copy(acc.at[pl.ds(s * NPT, NPT)],
                    out_hbm.at[c, pl.ds(s * NPT, NPT)])


def kernel(channels, edge_distances, edge_index, W1, b1, W2, b2):
    npad = E_PAD - E
    d_pad = jnp.concatenate([edge_distances, jnp.zeros((npad,), jnp.float32)])
    pad_i = jnp.arange(npad, dtype=jnp.int32)
    ctr = jnp.concatenate([edge_index[0], N + (pad_i % NS)])
    nbr = jnp.concatenate([edge_index[1], pad_i % NS])
    ctr2 = ctr.reshape(IDX_ROWS, CHUNK)
    nbr2 = nbr.reshape(IDX_ROWS, CHUNK)

    f_edge = _filter_call(d_pad.reshape(E_PAD // BE, BE), W1.T,
                          b1.reshape(HID, 1), W2, b2.reshape(1, D))
    partial = _sc_conv(channels, ctr2, nbr2, f_edge)
    return _add_call(partial)[:N]
